# Initial kernel scaffold; baseline (speedup 1.0000x reference)
import jax
import jax.numpy as jnp
from jax.experimental import pallas as pl


def kernel(x, edge_index, graph_ids, W1, b1, W2, b2, Wc1, bc1, Wc2, bc2, Wc3, bc3):
    raise NotImplementedError("write your pallas kernel here")


# speedup vs baseline: 13.4919x; 13.4919x over previous
import functools
import jax
import jax.numpy as jnp
from jax import lax
from jax.experimental import pallas as pl
from jax.experimental.pallas import tpu as pltpu
from jax.experimental.pallas import tpu_sc as plsc

N = 10000
E = 320000
G = 64
NPAD = 10240
NC, NS = 2, 16
ROWS_PER_TILE = NPAD // NS
CHUNK = 40

_mesh = plsc.VectorSubcoreMesh(core_axis_name="c", subcore_axis_name="s")


def _deg_body(ei_ref, ones_ref, zeros_ref, out_ref, hist, ones_v, idx_v, sem):
    cid = lax.axis_index("c")
    sid = lax.axis_index("s")
    r0 = sid * ROWS_PER_TILE
    pltpu.sync_copy(zeros_ref.at[pl.ds(r0, ROWS_PER_TILE)],
                    hist.at[pl.ds(r0, ROWS_PER_TILE)])
    pltpu.sync_copy(ones_ref, ones_v)
    plsc.subcore_barrier()

    n_chunks = E // NS // CHUNK
    pltpu.sync_copy(ei_ref.at[cid, sid], idx_v)

    W = 10

    @pl.loop(0, n_chunks, step=W)
    def _(g):
        for b in range(W):
            pltpu.async_copy(ones_v, hist.at[idx_v.at[g + b]], sem.at[b],
                             add=True)
        for b in range(W):
            pltpu.make_async_copy(ones_v, hist.at[idx_v.at[g + b]],
                                  sem.at[b]).wait()

    plsc.subcore_barrier()
    pltpu.sync_copy(hist.at[pl.ds(r0, ROWS_PER_TILE)],
                    out_ref.at[cid, pl.ds(r0, ROWS_PER_TILE)])


@jax.jit
def _degrees(ei4):
    ones = jnp.ones((CHUNK,), jnp.float32)
    zeros = jnp.zeros((NPAD,), jnp.float32)
    k = pl.kernel(
        _deg_body,
        out_type=jax.ShapeDtypeStruct((NC, NPAD), jnp.float32),
        mesh=_mesh,
        scratch_types=[
            pltpu.VMEM_SHARED((NPAD,), jnp.float32),
            pltpu.VMEM((CHUNK,), jnp.float32),
            pltpu.VMEM((E // NS // CHUNK, CHUNK), jnp.int32),
            pltpu.SemaphoreType.DMA((10,)),
        ],
    )
    return k(ei4, ones, zeros)


NBUF = 5


def _prop_body(p_ref, ei_ref, zeros_ref, out_ref,
               agg, sidx_v, didx_v, rows_v, gsem, ssem):
    cid = lax.axis_index("c")
    sid = lax.axis_index("s")
    r0 = sid * ROWS_PER_TILE
    pltpu.sync_copy(zeros_ref.at[pl.ds(r0, ROWS_PER_TILE)],
                    agg.at[pl.ds(r0, ROWS_PER_TILE)])
    wid = cid * NS + sid
    pltpu.sync_copy(ei_ref.at[0, wid], sidx_v)
    pltpu.sync_copy(ei_ref.at[1, wid], didx_v)
    plsc.subcore_barrier()

    n_chunks = E // (NC * NS) // CHUNK

    def gather(j, b):
        return pltpu.make_async_copy(p_ref.at[sidx_v.at[j]], rows_v.at[b],
                                     gsem.at[b])

    def scatter(j, b):
        return pltpu.make_async_copy(rows_v.at[b], agg.at[didx_v.at[j]],
                                     ssem.at[b])

    for b in range(NBUF):
        pltpu.async_copy(p_ref.at[sidx_v.at[b]], rows_v.at[b], gsem.at[b])

    @pl.loop(0, n_chunks - NBUF, step=NBUF)
    def _(g):
        for b in range(NBUF):
            gather(g + b, b).wait()
            pltpu.async_copy(rows_v.at[b], agg.at[didx_v.at[g + b]],
                             ssem.at[b], add=True)
        for b in range(NBUF):
            scatter(g + b, b).wait()
            pltpu.async_copy(p_ref.at[sidx_v.at[g + b + NBUF]], rows_v.at[b],
                             gsem.at[b])

    g0 = n_chunks - NBUF
    for b in range(NBUF):
        gather(g0 + b, b).wait()
        pltpu.async_copy(rows_v.at[b], agg.at[didx_v.at[g0 + b]],
                         ssem.at[b], add=True)
    for b in range(NBUF):
        scatter(g0 + b, b).wait()

    plsc.subcore_barrier()
    pltpu.sync_copy(agg.at[pl.ds(r0, ROWS_PER_TILE)],
                    out_ref.at[cid, pl.ds(r0, ROWS_PER_TILE)])


@functools.partial(jax.jit, static_argnames=("d",))
def _propagate(p, ei3, d):
    zeros = jnp.zeros((NPAD, d), jnp.float32)
    k = pl.kernel(
        _prop_body,
        out_type=jax.ShapeDtypeStruct((NC, NPAD, d), jnp.float32),
        mesh=_mesh,
        scratch_types=[
            pltpu.VMEM_SHARED((NPAD, d), jnp.float32),
            pltpu.VMEM((E // (NC * NS) // CHUNK, CHUNK), jnp.int32),
            pltpu.VMEM((E // (NC * NS) // CHUNK, CHUNK), jnp.int32),
            pltpu.VMEM((NBUF, CHUNK, d), jnp.float32),
            pltpu.SemaphoreType.DMA((NBUF,)),
            pltpu.SemaphoreType.DMA((NBUF,)),
        ],
        compiler_params=pltpu.CompilerParams(use_tc_tiling_on_sc=False),
    )
    return k(p, ei3, zeros)


def _pre1_body(x_ref, deg_ref, w_ref, o_ref):
    norm = lax.rsqrt(jnp.maximum(deg_ref[...], 1.0))
    o_ref[...] = jnp.dot(x_ref[...] * norm, w_ref[...],
                         preferred_element_type=jnp.float32,
                         precision=lax.Precision.HIGHEST)


def _mid_body(a0_ref, a1_ref, din_ref, dout_ref, b1_ref, w2_ref, o_ref):
    ndst = lax.rsqrt(jnp.maximum(din_ref[...], 1.0))
    nsrc = lax.rsqrt(jnp.maximum(dout_ref[...], 1.0))
    h1 = jnp.maximum((a0_ref[...] + a1_ref[...]) * ndst + b1_ref[...], 0.0)
    o_ref[...] = jnp.dot(h1 * nsrc, w2_ref[...],
                         preferred_element_type=jnp.float32,
                         precision=lax.Precision.HIGHEST)


def _post_body(a0_ref, a1_ref, din_ref, b2_ref, gid_ref,
               wc1_ref, bc1_ref, wc2_ref, bc2_ref, wc3_ref, bc3_ref, o_ref):
    ndst = lax.rsqrt(jnp.maximum(din_ref[...], 1.0))
    h2 = jnp.maximum((a0_ref[...] + a1_ref[...]) * ndst + b2_ref[...], 0.0)
    gid = gid_ref[...]
    rows = lax.broadcasted_iota(jnp.int32, (G, N), 0)
    onehot = (rows == gid).astype(jnp.float32)
    sums = jnp.dot(onehot, h2, preferred_element_type=jnp.float32,
                   precision=lax.Precision.HIGHEST)
    counts = jnp.sum(onehot, axis=1, keepdims=True)
    hg = sums / jnp.maximum(counts, 1.0)
    out = jnp.dot(hg, wc1_ref[...], preferred_element_type=jnp.float32,
                  precision=lax.Precision.HIGHEST) + bc1_ref[...]
    out = jnp.dot(out, wc2_ref[...], preferred_element_type=jnp.float32,
                  precision=lax.Precision.HIGHEST) + bc2_ref[...]
    out = jnp.dot(out, wc3_ref[...], preferred_element_type=jnp.float32,
                  precision=lax.Precision.HIGHEST) + bc3_ref[...]
    o_ref[...] = out


def _tc_call(body, out_shape, *args):
    return pl.pallas_call(body, out_shape=out_shape)(*args)


def kernel(x, edge_index, graph_ids, W1, b1, W2, b2,
           Wc1, bc1, Wc2, bc2, Wc3, bc3):
    ei = edge_index.astype(jnp.int32)
    ei_deg = ei.reshape(2, NS, E // NS // CHUNK, CHUNK)
    ei_prop = ei.reshape(2, NC * NS, E // (NC * NS) // CHUNK, CHUNK)

    degs = _degrees(ei_deg)
    deg_out = degs[0, :N].reshape(N, 1)
    deg_in = degs[1, :N].reshape(N, 1)

    p1 = _tc_call(_pre1_body, jax.ShapeDtypeStruct((N, W1.shape[1]), jnp.float32),
                  x, deg_out, W1)

    agg1 = _propagate(p1, ei_prop, W1.shape[1])
    p2 = _tc_call(_mid_body, jax.ShapeDtypeStruct((N, W2.shape[1]), jnp.float32),
                  agg1[0, :N], agg1[1, :N], deg_in, deg_out,
                  b1.reshape(1, -1), W2)

    agg2 = _propagate(p2, ei_prop, W2.shape[1])
    out = _tc_call(_post_body,
                   jax.ShapeDtypeStruct((G, Wc3.shape[1]), jnp.float32),
                   agg2[0, :N], agg2[1, :N], deg_in, b2.reshape(1, -1),
                   graph_ids.astype(jnp.int32).reshape(1, N),
                   Wc1, bc1.reshape(1, -1), Wc2, bc2.reshape(1, -1),
                   Wc3, bc3.reshape(1, -1))
    return out
